# Initial kernel scaffold; baseline (speedup 1.0000x reference)
import jax
import jax.numpy as jnp
from jax.experimental import pallas as pl


def kernel(latents, num_atoms, atom_types, gt_frac_coords, lengths, angles, batch, edge_index, time_steps, noise, sigmas, atom_emb, W_e1, b_e1, W_e2, b_e2, Ws1, bs1, Ws2, bs2, Ws3, bs3):
    raise NotImplementedError("write your pallas kernel here")


# speedup vs baseline: 1.0172x; 1.0172x over previous
import jax
import jax.numpy as jnp
from jax.experimental import pallas as pl
from jax.experimental.pallas import tpu as pltpu

EPS = 1e-8
CUTOFF = 5.0
NRBF = 16
B = 512


def _fin_rows(rows, rowmask):
    cf = rows[:, 7:8]
    c = jnp.where(cf > 0.0, cf, 1.0)
    t = rows[:, 0:3] / c - (rows[:, 3:6] / c) / ((rows[:, 6:7] / c) + EPS)
    return jnp.sum(t * t * rows[:, 8:9] * rowmask, axis=(0, 1), keepdims=True)


def _edge_kernel(delta_ref, locl_ref, edata_ref, T1j_ref, T1i_ref, W1r_ref,
                 be1_ref, We2_ref, be2_ref, Ws1h_ref, L1_ref, bs1_ref,
                 Ws2_ref, bs2_ref, Ws3_ref, bs3_ref, out_ref, carry_ref):
    pid = pl.program_id(0)
    nb = pl.num_programs(0)
    f32 = jnp.float32

    @pl.when(pid == 0)
    def _init():
        out_ref[...] = jnp.zeros((1, 1), f32)
        carry_ref[...] = jnp.zeros((8, 128), f32)

    ed = edata_ref[...]
    dv = ed[:, 0:3]
    gtdv = ed[:, 3:6]
    sig = ed[:, 6:7]
    msk = ed[:, 7:8]
    wcol = ed[:, 8:9]
    tj = ed[:, 9:10]
    ti = ed[:, 10:11]
    bj = ed[:, 11:12]
    loc = ed[:, 12:13]

    dist = jnp.sqrt(jnp.sum(dv * dv, axis=1, keepdims=True))
    centers = jax.lax.broadcasted_iota(jnp.int32, (B, NRBF), 1).astype(f32) \
        * (CUTOFF / (NRBF - 1))
    rbf = jnp.exp(-10.0 * (dist - centers) ** 2)

    i128 = jax.lax.broadcasted_iota(jnp.int32, (B, 128), 1).astype(f32)
    ohj = (tj == i128).astype(f32)
    ohi = (ti == i128).astype(f32)
    nlat = L1_ref.shape[0]
    ohb = (bj == jax.lax.broadcasted_iota(jnp.int32, (B, nlat), 1)
           .astype(f32)).astype(f32)

    def dot(a, b):
        return jax.lax.dot_general(a, b, (((1,), (0,)), ((), ())),
                                   preferred_element_type=f32)

    h1 = dot(ohj, T1j_ref[...]) + dot(ohi, T1i_ref[...]) \
        + dot(rbf, W1r_ref[...]) + be1_ref[...]
    h = dot(jnp.maximum(h1, 0.0), We2_ref[...]) + be2_ref[...]
    x1 = jnp.maximum(dot(h, Ws1h_ref[...]) + dot(ohb, L1_ref[...])
                     + bs1_ref[...], 0.0)
    x2 = jnp.maximum(dot(x1, Ws2_ref[...]) + bs2_ref[...], 0.0)
    scores = (dot(x2, Ws3_ref[...]) + bs3_ref[...])[:, 0:1]

    inv = 1.0 / (dist + EPS)
    vs = scores * dv * inv
    gt_d = jnp.sqrt(jnp.sum(gtdv * gtdv, axis=1, keepdims=True))
    vg = (gt_d - dist) * dv * inv
    val = jnp.concatenate(
        [msk * vs, msk * vg, msk * sig, msk, wcol, jnp.zeros((B, 7), f32)],
        axis=1)

    ohs = (loc == jax.lax.broadcasted_iota(jnp.int32, (B, B), 1)
           .astype(f32)).astype(f32)
    S = jax.lax.dot_general(ohs, val, (((0,), (0,)), ((), ())),
                            preferred_element_type=f32)

    delta = delta_ref[pid]
    locl = locl_ref[pid]
    rows_i = jax.lax.broadcasted_iota(jnp.int32, (B, 1), 0)
    dfac = jnp.where(delta == 0, 1.0, 0.0).astype(f32)
    crow = carry_ref[0:1, 0:16]
    S0 = S + crow * (rows_i == 0).astype(f32) * dfac

    contrib = _fin_rows(S0, (rows_i < locl).astype(f32))
    contrib = contrib + _fin_rows(crow, jnp.ones((1, 1), f32)) * (1.0 - dfac)
    new_carry = jnp.sum(S0 * (rows_i == locl).astype(f32), axis=0,
                        keepdims=True)
    lastfac = jnp.where(pid == nb - 1, 1.0, 0.0).astype(f32)
    contrib = contrib + _fin_rows(new_carry, jnp.ones((1, 1), f32)) * lastfac

    out_ref[...] += contrib
    carry_ref[0:1, 0:16] = new_carry


def kernel(latents, num_atoms, atom_types, gt_frac_coords, lengths, angles,
           batch, edge_index, time_steps, noise, sigmas, atom_emb,
           W_e1, b_e1, W_e2, b_e2, Ws1, bs1, Ws2, bs2, Ws3, bs3):
    f32 = jnp.float32
    Gn = latents.shape[0]
    N = gt_frac_coords.shape[0]
    E = edge_index.shape[1]
    A = atom_emb.shape[0]
    AE = atom_emb.shape[1]
    H = W_e2.shape[0]
    F = Ws2.shape[0]
    seg = jax.ops.segment_sum
    assert E % B == 0
    nb = E // B

    cnt = jnp.maximum(seg(jnp.ones((N,), f32), batch, Gn), 1.0)
    sig_n = sigmas[time_steps][batch]
    gt_cart = gt_frac_coords * lengths[batch]
    pert = gt_cart + sig_n[:, None] * noise
    pert = pert - (seg(pert, batch, Gn) / cnt[:, None])[batch]
    gt_cart = gt_cart - (seg(gt_cart, batch, Gn) / cnt[:, None])[batch]

    j = edge_index[0]
    i = edge_index[1]
    mask = i != j
    idx = jnp.arange(E, dtype=jnp.int32)
    pm = jax.lax.cummax(jnp.where(mask, idx, -1))
    prev = jnp.concatenate([jnp.full((1,), -1, pm.dtype), pm[:-1]])
    pc = jnp.maximum(prev, 0)
    start = mask & ((prev < 0) | (i != i[pc]) | (j != j[pc]))
    rid = jnp.cumsum(start.astype(jnp.int32)) - 1
    rid_f = jnp.maximum(jax.lax.cummax(jnp.where(mask, rid, -1)), 0)
    r0 = rid_f[::B]
    loc = rid_f - jnp.repeat(r0, B)
    locl = loc[B - 1::B].astype(jnp.int32)
    delta = jnp.concatenate(
        [jnp.zeros((1,), jnp.int32),
         (r0[1:] - (r0[:-1] + locl[:-1])).astype(jnp.int32)])
    node_cnt = seg(start.astype(f32), i, N)
    w_edge = jnp.where(start, 1.0 / jnp.maximum(node_cnt, 1.0)[i], 0.0)
    nnz = jnp.maximum(jnp.sum(node_cnt > 0.0), 1).astype(f32)

    dv = pert[i] - pert[j]
    gtdv = gt_cart[i] - gt_cart[j]
    edata = jnp.concatenate([
        dv, gtdv, sig_n[i][:, None], mask.astype(f32)[:, None],
        w_edge[:, None], atom_types[j].astype(f32)[:, None],
        atom_types[i].astype(f32)[:, None], batch[j].astype(f32)[:, None],
        loc.astype(f32)[:, None], jnp.zeros((E, 3), f32)], axis=1)

    T1j = jnp.zeros((128, H), f32).at[:A].set(atom_emb @ W_e1[:AE])
    T1i = jnp.zeros((128, H), f32).at[:A].set(atom_emb @ W_e1[AE:2 * AE])
    W1r = W_e1[2 * AE:]
    Ws1h = Ws1[:H]
    L1 = latents @ Ws1[H:]
    Ws3p = jnp.concatenate([Ws3, jnp.zeros((F, 7), f32)], axis=1)
    bs3p = jnp.concatenate([bs3, jnp.zeros((7,), f32)])

    const = lambda b: (0, 0)
    out = pl.pallas_call(
        _edge_kernel,
        grid=(nb,),
        in_specs=[
            pl.BlockSpec(memory_space=pltpu.SMEM),
            pl.BlockSpec(memory_space=pltpu.SMEM),
            pl.BlockSpec((B, 16), lambda b: (b, 0)),
            pl.BlockSpec((128, H), const),
            pl.BlockSpec((128, H), const),
            pl.BlockSpec((NRBF, H), const),
            pl.BlockSpec((1, H), const),
            pl.BlockSpec((H, H), const),
            pl.BlockSpec((1, H), const),
            pl.BlockSpec((H, F), const),
            pl.BlockSpec((Gn, F), const),
            pl.BlockSpec((1, F), const),
            pl.BlockSpec((F, F), const),
            pl.BlockSpec((1, F), const),
            pl.BlockSpec((F, 8), const),
            pl.BlockSpec((1, 8), const),
        ],
        out_specs=pl.BlockSpec((1, 1), const),
        out_shape=jax.ShapeDtypeStruct((1, 1), f32),
        scratch_shapes=[pltpu.VMEM((8, 128), f32)],
        compiler_params=pltpu.CompilerParams(
            dimension_semantics=("arbitrary",)),
    )(delta, locl, edata, T1j, T1i, W1r, b_e1[None, :], W_e2,
      b_e2[None, :], Ws1h, L1, bs1[None, :], Ws2, bs2[None, :], Ws3p,
      bs3p[None, :])

    return out[0, 0] / (nnz * 3.0)


# speedup vs baseline: 4.7542x; 4.6740x over previous
import functools
import jax
import jax.numpy as jnp
from jax import lax
from jax.experimental import pallas as pl
from jax.experimental.pallas import tpu as pltpu
from jax.experimental.pallas import tpu_sc as plsc

EPS = 1e-8
CUTOFF = 5.0
NRBF = 16
B = 512
CHUNK = 200


def _gather_rows_sc(ntab, i_idx, j_idx):
    f32 = jnp.float32
    i32 = jnp.int32
    E = i_idx.shape[0]
    info = plsc.get_sparse_core_info()
    NW = info.num_cores * info.num_subcores
    per_w = E // NW
    assert E % NW == 0 and per_w % CHUNK == 0
    n_chunks = per_w // CHUNK
    npages = NW * n_chunks

    mesh = plsc.VectorSubcoreMesh(core_axis_name="c", subcore_axis_name="s")

    @functools.partial(
        pl.kernel, mesh=mesh,
        out_type=(jax.ShapeDtypeStruct((npages, CHUNK, 128), f32),
                  jax.ShapeDtypeStruct((npages, CHUNK, 128), f32)),
        scratch_types=[
            pltpu.VMEM((CHUNK,), i32),
            pltpu.VMEM((CHUNK,), i32),
            pltpu.VMEM((CHUNK, 128), f32),
            pltpu.VMEM((CHUNK, 128), f32),
            pltpu.SemaphoreType.DMA,
        ],
    )
    def _sc(tab_hbm, ih, jh, oi_hbm, oj_hbm, iv, jv, ri, rj, sem):
        wid = lax.axis_index("s") * info.num_cores + lax.axis_index("c")
        base = wid * per_w

        def outer(ci, _):
            cbase = base + ci * CHUNK
            pg = wid * n_chunks + ci
            pltpu.sync_copy(ih.at[pl.ds(cbase, CHUNK)], iv)
            pltpu.sync_copy(jh.at[pl.ds(cbase, CHUNK)], jv)
            pltpu.async_copy(tab_hbm.at[iv], ri, sem).wait()
            pltpu.async_copy(tab_hbm.at[jv], rj, sem).wait()
            pltpu.sync_copy(ri, oi_hbm.at[pg])
            pltpu.sync_copy(rj, oj_hbm.at[pg])
            return 0
        lax.fori_loop(0, n_chunks, outer, 0)

    oi, oj = _sc(ntab, i_idx.astype(i32), j_idx.astype(i32))
    return oi.reshape(E, 128), oj.reshape(E, 128)


def _fin_rows(rows, rowmask):
    cf = rows[:, 7:8]
    c = jnp.where(cf > 0.0, cf, 1.0)
    t = rows[:, 0:3] / c - (rows[:, 3:6] / c) / ((rows[:, 6:7] / c) + EPS)
    return jnp.sum(t * t * rows[:, 8:9] * rowmask, axis=(0, 1), keepdims=True)


def _edge_kernel(delta_ref, locl_ref, edi_ref, edj_ref, wl_ref, T1j_ref,
                 T1i_ref, W1r_ref, be1_ref, We2_ref, be2_ref, Ws1h_ref,
                 L1_ref, bs1_ref, Ws2_ref, bs2_ref, Ws3_ref, bs3_ref,
                 out_ref, carry_ref):
    pid = pl.program_id(0)
    nb = pl.num_programs(0)
    f32 = jnp.float32

    @pl.when(pid == 0)
    def _init():
        out_ref[...] = jnp.zeros((1, 1), f32)
        carry_ref[...] = jnp.zeros((8, 128), f32)

    edi = edi_ref[...]
    edj = edj_ref[...]
    wl = wl_ref[...]
    d = edi - edj
    dv = d[:, 0:3]
    gtdv = d[:, 3:6]
    sig = edi[:, 6:7]
    msk = (d[:, 7:8] != 0.0).astype(f32)
    tj = edj[:, 9:10]
    ti = edi[:, 10:11]
    bj = edj[:, 11:12]
    wcol = wl[:, 0:1]
    loc = wl[:, 1:2]

    dist = jnp.sqrt(jnp.sum(dv * dv, axis=1, keepdims=True))
    centers = jax.lax.broadcasted_iota(jnp.int32, (B, NRBF), 1).astype(f32) \
        * (CUTOFF / (NRBF - 1))
    rbf = jnp.exp(-10.0 * (dist - centers) ** 2)

    i128 = jax.lax.broadcasted_iota(jnp.int32, (B, 128), 1).astype(f32)
    ohj = (tj == i128).astype(f32)
    ohi = (ti == i128).astype(f32)
    nlat = L1_ref.shape[0]
    ohb = (bj == jax.lax.broadcasted_iota(jnp.int32, (B, nlat), 1)
           .astype(f32)).astype(f32)

    def dot(a, b):
        return jax.lax.dot_general(a, b, (((1,), (0,)), ((), ())),
                                   preferred_element_type=f32)

    h1 = dot(ohj, T1j_ref[...]) + dot(ohi, T1i_ref[...]) \
        + dot(rbf, W1r_ref[...]) + be1_ref[...]
    h = dot(jnp.maximum(h1, 0.0), We2_ref[...]) + be2_ref[...]
    x1 = jnp.maximum(dot(h, Ws1h_ref[...]) + dot(ohb, L1_ref[...])
                     + bs1_ref[...], 0.0)
    x2 = jnp.maximum(dot(x1, Ws2_ref[...]) + bs2_ref[...], 0.0)
    scores = (dot(x2, Ws3_ref[...]) + bs3_ref[...])[:, 0:1]

    inv = 1.0 / (dist + EPS)
    vs = scores * dv * inv
    gt_d = jnp.sqrt(jnp.sum(gtdv * gtdv, axis=1, keepdims=True))
    vg = (gt_d - dist) * dv * inv
    val = jnp.concatenate(
        [msk * vs, msk * vg, msk * sig, msk, wcol, jnp.zeros((B, 7), f32)],
        axis=1)

    ohs = (loc == jax.lax.broadcasted_iota(jnp.int32, (B, B), 1)
           .astype(f32)).astype(f32)
    S = jax.lax.dot_general(ohs, val, (((0,), (0,)), ((), ())),
                            preferred_element_type=f32)

    delta = delta_ref[pid]
    locl = locl_ref[pid]
    rows_i = jax.lax.broadcasted_iota(jnp.int32, (B, 1), 0)
    dfac = jnp.where(delta == 0, 1.0, 0.0).astype(f32)
    crow = carry_ref[0:1, 0:16]
    S0 = S + crow * (rows_i == 0).astype(f32) * dfac

    contrib = _fin_rows(S0, (rows_i < locl).astype(f32))
    contrib = contrib + _fin_rows(crow, jnp.ones((1, 1), f32)) * (1.0 - dfac)
    new_carry = jnp.sum(S0 * (rows_i == locl).astype(f32), axis=0,
                        keepdims=True)
    lastfac = jnp.where(pid == nb - 1, 1.0, 0.0).astype(f32)
    contrib = contrib + _fin_rows(new_carry, jnp.ones((1, 1), f32)) * lastfac

    out_ref[...] += contrib
    carry_ref[0:1, 0:16] = new_carry


def kernel(latents, num_atoms, atom_types, gt_frac_coords, lengths, angles,
           batch, edge_index, time_steps, noise, sigmas, atom_emb,
           W_e1, b_e1, W_e2, b_e2, Ws1, bs1, Ws2, bs2, Ws3, bs3):
    f32 = jnp.float32
    Gn = latents.shape[0]
    N = gt_frac_coords.shape[0]
    E = edge_index.shape[1]
    A = atom_emb.shape[0]
    AE = atom_emb.shape[1]
    H = W_e2.shape[0]
    F = Ws2.shape[0]
    seg = jax.ops.segment_sum
    assert E % B == 0
    nb = E // B

    cnt = jnp.maximum(seg(jnp.ones((N,), f32), batch, Gn), 1.0)
    sigg = sigmas[time_steps]
    sig_n = sigg[batch]
    gt_cart = gt_frac_coords * lengths[batch]
    pert = gt_cart + sig_n[:, None] * noise
    pert = pert - (seg(pert, batch, Gn) / cnt[:, None])[batch]
    gt_cart = gt_cart - (seg(gt_cart, batch, Gn) / cnt[:, None])[batch]

    j = edge_index[0]
    i = edge_index[1]
    mask = i != j
    idx = jnp.arange(E, dtype=jnp.int32)
    pm = jax.lax.cummax(jnp.where(mask, idx, -1))
    prev = jnp.concatenate([jnp.full((1,), -1, pm.dtype), pm[:-1]])
    pc = jnp.maximum(prev, 0)
    start = mask & ((prev < 0) | (i != i[pc]) | (j != j[pc]))
    rid = jnp.cumsum(start.astype(jnp.int32)) - 1
    rid_f = jnp.maximum(jax.lax.cummax(jnp.where(mask, rid, -1)), 0)
    r0 = rid_f[::B]
    loc = rid_f - jnp.repeat(r0, B)
    locl = loc[B - 1::B].astype(jnp.int32)
    delta = jnp.concatenate(
        [jnp.zeros((1,), jnp.int32),
         (r0[1:] - (r0[:-1] + locl[:-1])).astype(jnp.int32)])
    node_cnt = seg(start.astype(f32), i, N)
    w_edge = jnp.where(start, 1.0 / jnp.maximum(node_cnt, 1.0)[i], 0.0)
    nnz = jnp.maximum(jnp.sum(node_cnt > 0.0), 1).astype(f32)

    typef = atom_types.astype(f32)[:, None]
    ntab = jnp.concatenate([
        pert, gt_cart, sig_n[:, None],
        jnp.arange(N, dtype=f32)[:, None], jnp.zeros((N, 1), f32),
        typef, typef, batch.astype(f32)[:, None],
        jnp.zeros((N, 116), f32)], axis=1)
    edi, edj = _gather_rows_sc(ntab, i, j)
    wl = jnp.stack([w_edge, loc.astype(f32)], axis=1)

    T1j = jnp.zeros((128, H), f32).at[:A].set(atom_emb @ W_e1[:AE])
    T1i = jnp.zeros((128, H), f32).at[:A].set(atom_emb @ W_e1[AE:2 * AE])
    W1r = W_e1[2 * AE:]
    Ws1h = Ws1[:H]
    L1 = latents @ Ws1[H:]
    Ws3p = jnp.concatenate([Ws3, jnp.zeros((F, 7), f32)], axis=1)
    bs3p = jnp.concatenate([bs3, jnp.zeros((7,), f32)])

    const = lambda b: (0, 0)
    out = pl.pallas_call(
        _edge_kernel,
        grid=(nb,),
        in_specs=[
            pl.BlockSpec(memory_space=pltpu.SMEM),
            pl.BlockSpec(memory_space=pltpu.SMEM),
            pl.BlockSpec((B, 128), lambda b: (b, 0)),
            pl.BlockSpec((B, 128), lambda b: (b, 0)),
            pl.BlockSpec((B, 2), lambda b: (b, 0)),
            pl.BlockSpec((128, H), const),
            pl.BlockSpec((128, H), const),
            pl.BlockSpec((NRBF, H), const),
            pl.BlockSpec((1, H), const),
            pl.BlockSpec((H, H), const),
            pl.BlockSpec((1, H), const),
            pl.BlockSpec((H, F), const),
            pl.BlockSpec((Gn, F), const),
            pl.BlockSpec((1, F), const),
            pl.BlockSpec((F, F), const),
            pl.BlockSpec((1, F), const),
            pl.BlockSpec((F, 8), const),
            pl.BlockSpec((1, 8), const),
        ],
        out_specs=pl.BlockSpec((1, 1), const),
        out_shape=jax.ShapeDtypeStruct((1, 1), f32),
        scratch_shapes=[pltpu.VMEM((8, 128), f32)],
        compiler_params=pltpu.CompilerParams(
            dimension_semantics=("arbitrary",)),
    )(delta, locl, edi, edj, wl, T1j, T1i, W1r, b_e1[None, :], W_e2,
      b_e2[None, :], Ws1h, L1, bs1[None, :], Ws2, bs2[None, :], Ws3p,
      bs3p[None, :])

    return out[0, 0] / (nnz * 3.0)
